# baseline (device time: 138689 ns/iter reference)
import jax
import jax.numpy as jnp
import numpy as np
from jax import lax
from jax.experimental import pallas as pl
from jax.experimental.pallas import tpu as pltpu

N_DEV = 4
SQ = 2048
SKV = 2048
H_LOC = 8
DH = 128
D_MODEL = 1024
D_LOC = H_LOC * DH
SCALE = 0.08838834764831843
QB = 64
NQB = SQ // QB
CHUNK = SQ // N_DEV

QBS = [[qb for qb in range(NQB) if qb % 3 == c] for c in range(3)]
KBS = []
for c in range(3):
    ks = [kb for kb in range(NQB) if kb % 3 == (3 - c) % 3]
    if 0 not in ks:
        ks = [0] + ks
    KBS.append(ks)
NK = [len(k) * QB for k in KBS]

PERM_BLOCKS = QBS[0] + QBS[1] + QBS[2]
INV_PERM = list(np.argsort(PERM_BLOCKS))
SEGMENTS = {
    0: [(0, 8, 0, 0)],
    1: [(0, 3, 0, 8), (192, 5, 1, 0)],
    2: [(0, 6, 1, 5), (384, 2, 2, 0)],
    3: [(0, 8, 2, 2)],
}


def _body(xp_ref, wq_ref, wo_ref, k_ref, v_ref, out_ref,
          q_ref, ctx_ref, ks0, ks1, ks2, vv0, vv1, vv2,
          kd1, kd2, vd1, vd2, dbias_ref,
          rs_send, rs_recv, ag_send, ag_recv,
          ss_rs, sr_rs, ss_ag, sr_ag):
    ksub = [ks0, ks1, ks2]
    vsub = [vv0, vv1, vv2]
    kdiag = [None, kd1, kd2]
    vdiag = [None, vd1, vd2]

    my_pos = lax.axis_index("i")

    barrier_sem = pltpu.get_barrier_semaphore()
    for d in range(1, N_DEV):
        pl.semaphore_signal(
            barrier_sem, inc=1,
            device_id=((my_pos + d) % N_DEV,),
            device_id_type=pl.DeviceIdType.MESH,
        )
    pl.semaphore_wait(barrier_sem, N_DEV - 1)

    for c in range(3):
        for i, b in enumerate(KBS[c]):
            ksub[c][i * QB:(i + 1) * QB, :] = k_ref[b * QB:(b + 1) * QB, :]
            vsub[c][i * QB:(i + 1) * QB, :] = v_ref[b * QB:(b + 1) * QB, :]
    for c in (1, 2):
        for i, b in enumerate(QBS[c]):
            kdiag[c][i * QB:(i + 1) * QB, :] = k_ref[b * QB:(b + 1) * QB, :]
            vdiag[c][i * QB:(i + 1) * QB, :] = v_ref[b * QB:(b + 1) * QB, :]
    ib = lax.broadcasted_iota(jnp.int32, (CHUNK, CHUNK), 0) // QB
    jb = lax.broadcasted_iota(jnp.int32, (CHUNK, CHUNK), 1) // QB
    dbias_ref[...] = jnp.where(ib == jb, 0.0, -1e9).astype(jnp.bfloat16)

    def hcols(h):
        return slice(h * DH, (h + 1) * DH)

    def compute_chunk(k):
        for (off, nb, rc, coff) in SEGMENTS[k]:
            m = nb * QB
            rows = slice(k * CHUNK + off, k * CHUNK + off + m)
            seg = slice(off, off + m)
            q_ref[seg, :] = jnp.dot(
                xp_ref[rows, :], wq_ref[...],
                preferred_element_type=jnp.float32,
            ).astype(jnp.bfloat16)
            for h in range(H_LOC):
                q = q_ref[seg, hcols(h)]
                sm = lax.dot_general(
                    q, ksub[rc][:, hcols(h)], (((1,), (1,)), ((), ())),
                    preferred_element_type=jnp.float32,
                )
                em = jnp.exp(sm)
                s = jnp.sum(em, axis=1, keepdims=True)
                if rc != 0:
                    dcols = slice(coff * QB, (coff + nb) * QB)
                    sd = lax.dot_general(
                        q, kdiag[rc][dcols, hcols(h)],
                        (((1,), (1,)), ((), ())),
                        preferred_element_type=jnp.float32,
                    ) + dbias_ref[:m, :nb * QB].astype(jnp.float32)
                    ed = jnp.exp(sd)
                    s = s + jnp.sum(ed, axis=1, keepdims=True)
                inv = 1.0 / s
                ctxh = jnp.dot(
                    (em * inv).astype(jnp.bfloat16), vsub[rc][:, hcols(h)],
                    preferred_element_type=jnp.float32,
                )
                if rc != 0:
                    ctxh = ctxh + jnp.dot(
                        (ed * inv).astype(jnp.bfloat16),
                        vdiag[rc][dcols, hcols(h)],
                        preferred_element_type=jnp.float32,
                    )
                ctx_ref[seg, hcols(h)] = ctxh.astype(jnp.bfloat16)
        return jnp.dot(
            ctx_ref[...], wo_ref[...], preferred_element_type=jnp.float32
        )

    for k in range(N_DEV):
        p = compute_chunk(k)
        owner = (k - 1) % N_DEV
        rel = (my_pos - owner) % N_DEV
        rs_send[k] = p.astype(jnp.bfloat16)

        @pl.when(my_pos == owner)
        def _(k=k):
            out_ref[k * CHUNK:(k + 1) * CHUNK, :] = rs_send[k]

        for r in range(1, N_DEV):
            @pl.when(rel == r)
            def _(k=k, r=r, owner=owner):
                pltpu.make_async_remote_copy(
                    src_ref=rs_send.at[k],
                    dst_ref=rs_recv.at[r - 1],
                    send_sem=ss_rs.at[k],
                    recv_sem=sr_rs.at[r - 1],
                    device_id=(owner,),
                    device_id_type=pl.DeviceIdType.MESH,
                ).start()

    def recv_only(dst, sem):
        return pltpu.make_async_remote_copy(
            src_ref=rs_send.at[0], dst_ref=dst,
            send_sem=ss_ag.at[0], recv_sem=sem,
            device_id=(my_pos,), device_id_type=pl.DeviceIdType.MESH,
        )

    for d in range(3):
        recv_only(rs_recv.at[d], sr_rs.at[d]).wait_recv()
    srows = pl.ds(((my_pos + 1) % N_DEV) * CHUNK, CHUNK)
    out_ref[srows, :] = (
        out_ref[srows, :].astype(jnp.float32)
        + rs_recv[0].astype(jnp.float32)
        + rs_recv[1].astype(jnp.float32)
        + rs_recv[2].astype(jnp.float32)
    ).astype(jnp.bfloat16)

    ag_send[...] = out_ref[srows, :]
    ag_sends = []
    for i, (dslot, dev_off) in enumerate([(2, 1), (1, 2), (0, 3)]):
        r = pltpu.make_async_remote_copy(
            src_ref=ag_send,
            dst_ref=ag_recv.at[dslot],
            send_sem=ss_ag.at[i],
            recv_sem=sr_ag.at[dslot],
            device_id=((my_pos + dev_off) % N_DEV,),
            device_id_type=pl.DeviceIdType.MESH,
        )
        r.start()
        ag_sends.append(r)
    for slot, chunk_off in [(0, 2), (2, 0), (1, 3)]:
        recv_only(ag_recv.at[slot], sr_ag.at[slot]).wait_recv()
        rows = pl.ds(((my_pos + chunk_off) % N_DEV) * CHUNK, CHUNK)
        out_ref[rows, :] = ag_recv[slot]

    for k in range(N_DEV):
        @pl.when(my_pos != (k - 1) % N_DEV)
        def _(k=k):
            pltpu.make_async_remote_copy(
                src_ref=rs_send.at[k], dst_ref=rs_recv.at[0],
                send_sem=ss_rs.at[k], recv_sem=sr_rs.at[0],
                device_id=(my_pos,), device_id_type=pl.DeviceIdType.MESH,
            ).wait_send()
    for r in ag_sends:
        r.wait_send()


def kernel(x, Wq, K_ext, V_ext, Wo):
    my_pos = lax.axis_index("i")
    wq_loc = (
        lax.dynamic_slice(Wq, (0, my_pos * D_LOC), (Wq.shape[0], D_LOC))
        * SCALE
    ).astype(jnp.bfloat16)
    wo_loc = lax.dynamic_slice(
        Wo, (my_pos * D_LOC, 0), (D_LOC, Wo.shape[1])
    ).astype(jnp.bfloat16)

    xb = x[0].astype(jnp.bfloat16).reshape(NQB, QB, x.shape[2])
    xp = jnp.take(xb, jnp.array(PERM_BLOCKS), axis=0).reshape(SQ, x.shape[2])

    k2 = K_ext[0].astype(jnp.bfloat16).reshape(SKV, D_LOC)
    v2 = V_ext[0].astype(jnp.bfloat16).reshape(SKV, D_LOC)

    out_p = pl.pallas_call(
        _body,
        out_shape=jax.ShapeDtypeStruct((SQ, D_MODEL), jnp.bfloat16),
        in_specs=[pl.BlockSpec(memory_space=pltpu.VMEM)] * 5,
        out_specs=pl.BlockSpec(memory_space=pltpu.VMEM),
        scratch_shapes=[
            pltpu.VMEM((CHUNK, D_LOC), jnp.bfloat16),
            pltpu.VMEM((CHUNK, D_LOC), jnp.bfloat16),
            pltpu.VMEM((NK[0], D_LOC), jnp.bfloat16),
            pltpu.VMEM((NK[1], D_LOC), jnp.bfloat16),
            pltpu.VMEM((NK[2], D_LOC), jnp.bfloat16),
            pltpu.VMEM((NK[0], D_LOC), jnp.bfloat16),
            pltpu.VMEM((NK[1], D_LOC), jnp.bfloat16),
            pltpu.VMEM((NK[2], D_LOC), jnp.bfloat16),
            pltpu.VMEM((len(QBS[1]) * QB, D_LOC), jnp.bfloat16),
            pltpu.VMEM((len(QBS[2]) * QB, D_LOC), jnp.bfloat16),
            pltpu.VMEM((len(QBS[1]) * QB, D_LOC), jnp.bfloat16),
            pltpu.VMEM((len(QBS[2]) * QB, D_LOC), jnp.bfloat16),
            pltpu.VMEM((CHUNK, CHUNK), jnp.bfloat16),
            pltpu.VMEM((N_DEV, CHUNK, D_MODEL), jnp.bfloat16),
            pltpu.VMEM((3, CHUNK, D_MODEL), jnp.bfloat16),
            pltpu.VMEM((CHUNK, D_MODEL), jnp.bfloat16),
            pltpu.VMEM((3, CHUNK, D_MODEL), jnp.bfloat16),
            pltpu.SemaphoreType.DMA((N_DEV,)),
            pltpu.SemaphoreType.DMA((3,)),
            pltpu.SemaphoreType.DMA((3,)),
            pltpu.SemaphoreType.DMA((3,)),
        ],
        compiler_params=pltpu.CompilerParams(
            collective_id=0,
            vmem_limit_bytes=62 * 1024 * 1024,
        ),
    )(xp, wq_loc, wo_loc, k2, v2)

    out = jnp.take(
        out_p.reshape(NQB, QB, D_MODEL), jnp.array(INV_PERM), axis=0
    ).reshape(1, SQ, D_MODEL)
    return out


# device time: 132176 ns/iter; 1.0493x vs baseline; 1.0493x over previous
import jax
import jax.numpy as jnp
import numpy as np
from jax import lax
from jax.experimental import pallas as pl
from jax.experimental.pallas import tpu as pltpu

N_DEV = 4
SQ = 2048
SKV = 2048
H_LOC = 8
DH = 128
D_MODEL = 1024
D_LOC = H_LOC * DH
SCALE = 0.08838834764831843
QB = 64
NQB = SQ // QB
CHUNK = SQ // N_DEV

QBS = [[qb for qb in range(NQB) if qb % 3 == c] for c in range(3)]
KBS = []
for c in range(3):
    ks = [kb for kb in range(NQB) if kb % 3 == (3 - c) % 3]
    if 0 not in ks:
        ks = [0] + ks
    KBS.append(ks)
NK = [len(k) * QB for k in KBS]

PERM_BLOCKS = QBS[0] + QBS[1] + QBS[2]
INV_PERM = list(np.argsort(PERM_BLOCKS))
SEGMENTS = {
    0: [(0, 8, 0, 0)],
    1: [(0, 3, 0, 8), (192, 5, 1, 0)],
    2: [(0, 6, 1, 5), (384, 2, 2, 0)],
    3: [(0, 8, 2, 2)],
}


def _body(xp_ref, wq_ref, wo_ref, k_ref, v_ref, out_ref,
          q_ref, ctx_ref, ks0, ks1, ks2, vv0, vv1, vv2,
          kd1, kd2, vd1, vd2, dbias_ref,
          rs_send, rs_recv, ag_send, ag_recv,
          ss_rs, sr_rs, ss_ag, sr_ag):
    ksub = [ks0, ks1, ks2]
    vsub = [vv0, vv1, vv2]
    kdiag = [None, kd1, kd2]
    vdiag = [None, vd1, vd2]

    my_pos = lax.axis_index("i")

    barrier_sem = pltpu.get_barrier_semaphore()
    for d in range(1, N_DEV):
        pl.semaphore_signal(
            barrier_sem, inc=1,
            device_id=((my_pos + d) % N_DEV,),
            device_id_type=pl.DeviceIdType.MESH,
        )
    pl.semaphore_wait(barrier_sem, N_DEV - 1)

    for c in range(3):
        for i, b in enumerate(KBS[c]):
            ksub[c][i * QB:(i + 1) * QB, :] = k_ref[b * QB:(b + 1) * QB, :]
            vsub[c][i * QB:(i + 1) * QB, :] = v_ref[b * QB:(b + 1) * QB, :]
    for c in (1, 2):
        for i, b in enumerate(QBS[c]):
            kdiag[c][i * QB:(i + 1) * QB, :] = k_ref[b * QB:(b + 1) * QB, :]
            vdiag[c][i * QB:(i + 1) * QB, :] = v_ref[b * QB:(b + 1) * QB, :]
    ib = lax.broadcasted_iota(jnp.int32, (CHUNK, CHUNK), 0) // QB
    jb = lax.broadcasted_iota(jnp.int32, (CHUNK, CHUNK), 1) // QB
    dbias_ref[...] = jnp.where(ib == jb, 0.0, -1e9).astype(jnp.bfloat16)

    def hcols(h):
        return slice(h * DH, (h + 1) * DH)

    def compute_chunk(k):
        for (off, nb, rc, coff) in SEGMENTS[k]:
            m = nb * QB
            rows = slice(k * CHUNK + off, k * CHUNK + off + m)
            seg = slice(off, off + m)
            q_ref[seg, :] = jnp.dot(
                xp_ref[rows, :], wq_ref[...],
                preferred_element_type=jnp.float32,
            ).astype(jnp.bfloat16)
            for h in range(H_LOC):
                q = q_ref[seg, hcols(h)]
                sm = lax.dot_general(
                    q, ksub[rc][:, hcols(h)], (((1,), (1,)), ((), ())),
                    preferred_element_type=jnp.float32,
                )
                em = jnp.exp(sm)
                s = jnp.sum(em, axis=1, keepdims=True)
                if rc != 0:
                    dcols = slice(coff * QB, (coff + nb) * QB)
                    sd = lax.dot_general(
                        q, kdiag[rc][dcols, hcols(h)],
                        (((1,), (1,)), ((), ())),
                        preferred_element_type=jnp.float32,
                    ) + dbias_ref[:m, :nb * QB].astype(jnp.float32)
                    ed = jnp.exp(sd)
                    s = s + jnp.sum(ed, axis=1, keepdims=True)
                ctxh = jnp.dot(
                    em.astype(jnp.bfloat16), vsub[rc][:, hcols(h)],
                    preferred_element_type=jnp.float32,
                )
                if rc != 0:
                    ctxh = ctxh + jnp.dot(
                        ed.astype(jnp.bfloat16),
                        vdiag[rc][dcols, hcols(h)],
                        preferred_element_type=jnp.float32,
                    )
                ctx_ref[seg, hcols(h)] = (ctxh * (1.0 / s)).astype(
                    jnp.bfloat16
                )
        return jnp.dot(
            ctx_ref[...], wo_ref[...], preferred_element_type=jnp.float32
        )

    for k in range(N_DEV):
        p = compute_chunk(k)
        owner = (k - 1) % N_DEV
        rel = (my_pos - owner) % N_DEV
        rs_send[k] = p.astype(jnp.bfloat16)

        @pl.when(my_pos == owner)
        def _(k=k):
            out_ref[k * CHUNK:(k + 1) * CHUNK, :] = rs_send[k]

        for r in range(1, N_DEV):
            @pl.when(rel == r)
            def _(k=k, r=r, owner=owner):
                pltpu.make_async_remote_copy(
                    src_ref=rs_send.at[k],
                    dst_ref=rs_recv.at[r - 1],
                    send_sem=ss_rs.at[k],
                    recv_sem=sr_rs.at[r - 1],
                    device_id=(owner,),
                    device_id_type=pl.DeviceIdType.MESH,
                ).start()

    def recv_only(dst, sem):
        return pltpu.make_async_remote_copy(
            src_ref=rs_send.at[0], dst_ref=dst,
            send_sem=ss_ag.at[0], recv_sem=sem,
            device_id=(my_pos,), device_id_type=pl.DeviceIdType.MESH,
        )

    for d in range(3):
        recv_only(rs_recv.at[d], sr_rs.at[d]).wait_recv()
    srows = pl.ds(((my_pos + 1) % N_DEV) * CHUNK, CHUNK)
    out_ref[srows, :] = (
        out_ref[srows, :].astype(jnp.float32)
        + rs_recv[0].astype(jnp.float32)
        + rs_recv[1].astype(jnp.float32)
        + rs_recv[2].astype(jnp.float32)
    ).astype(jnp.bfloat16)

    ag_send[...] = out_ref[srows, :]
    ag_sends = []
    for i, (dslot, dev_off) in enumerate([(2, 1), (1, 2), (0, 3)]):
        r = pltpu.make_async_remote_copy(
            src_ref=ag_send,
            dst_ref=ag_recv.at[dslot],
            send_sem=ss_ag.at[i],
            recv_sem=sr_ag.at[dslot],
            device_id=((my_pos + dev_off) % N_DEV,),
            device_id_type=pl.DeviceIdType.MESH,
        )
        r.start()
        ag_sends.append(r)
    for slot, chunk_off in [(0, 2), (2, 0), (1, 3)]:
        recv_only(ag_recv.at[slot], sr_ag.at[slot]).wait_recv()
        rows = pl.ds(((my_pos + chunk_off) % N_DEV) * CHUNK, CHUNK)
        out_ref[rows, :] = ag_recv[slot]

    for k in range(N_DEV):
        @pl.when(my_pos != (k - 1) % N_DEV)
        def _(k=k):
            pltpu.make_async_remote_copy(
                src_ref=rs_send.at[k], dst_ref=rs_recv.at[0],
                send_sem=ss_rs.at[k], recv_sem=sr_rs.at[0],
                device_id=(my_pos,), device_id_type=pl.DeviceIdType.MESH,
            ).wait_send()
    for r in ag_sends:
        r.wait_send()


def kernel(x, Wq, K_ext, V_ext, Wo):
    my_pos = lax.axis_index("i")
    wq_loc = (
        lax.dynamic_slice(Wq, (0, my_pos * D_LOC), (Wq.shape[0], D_LOC))
        * SCALE
    ).astype(jnp.bfloat16)
    wo_loc = lax.dynamic_slice(
        Wo, (my_pos * D_LOC, 0), (D_LOC, Wo.shape[1])
    ).astype(jnp.bfloat16)

    xb = x[0].astype(jnp.bfloat16).reshape(NQB, QB, x.shape[2])
    xp = jnp.take(xb, jnp.array(PERM_BLOCKS), axis=0).reshape(SQ, x.shape[2])

    k2 = K_ext[0].astype(jnp.bfloat16).reshape(SKV, D_LOC)
    v2 = V_ext[0].astype(jnp.bfloat16).reshape(SKV, D_LOC)

    out_p = pl.pallas_call(
        _body,
        out_shape=jax.ShapeDtypeStruct((SQ, D_MODEL), jnp.bfloat16),
        in_specs=[pl.BlockSpec(memory_space=pltpu.VMEM)] * 5,
        out_specs=pl.BlockSpec(memory_space=pltpu.VMEM),
        scratch_shapes=[
            pltpu.VMEM((CHUNK, D_LOC), jnp.bfloat16),
            pltpu.VMEM((CHUNK, D_LOC), jnp.bfloat16),
            pltpu.VMEM((NK[0], D_LOC), jnp.bfloat16),
            pltpu.VMEM((NK[1], D_LOC), jnp.bfloat16),
            pltpu.VMEM((NK[2], D_LOC), jnp.bfloat16),
            pltpu.VMEM((NK[0], D_LOC), jnp.bfloat16),
            pltpu.VMEM((NK[1], D_LOC), jnp.bfloat16),
            pltpu.VMEM((NK[2], D_LOC), jnp.bfloat16),
            pltpu.VMEM((len(QBS[1]) * QB, D_LOC), jnp.bfloat16),
            pltpu.VMEM((len(QBS[2]) * QB, D_LOC), jnp.bfloat16),
            pltpu.VMEM((len(QBS[1]) * QB, D_LOC), jnp.bfloat16),
            pltpu.VMEM((len(QBS[2]) * QB, D_LOC), jnp.bfloat16),
            pltpu.VMEM((CHUNK, CHUNK), jnp.bfloat16),
            pltpu.VMEM((N_DEV, CHUNK, D_MODEL), jnp.bfloat16),
            pltpu.VMEM((3, CHUNK, D_MODEL), jnp.bfloat16),
            pltpu.VMEM((CHUNK, D_MODEL), jnp.bfloat16),
            pltpu.VMEM((3, CHUNK, D_MODEL), jnp.bfloat16),
            pltpu.SemaphoreType.DMA((N_DEV,)),
            pltpu.SemaphoreType.DMA((3,)),
            pltpu.SemaphoreType.DMA((3,)),
            pltpu.SemaphoreType.DMA((3,)),
        ],
        compiler_params=pltpu.CompilerParams(
            collective_id=0,
            vmem_limit_bytes=62 * 1024 * 1024,
        ),
    )(xp, wq_loc, wo_loc, k2, v2)

    out = jnp.take(
        out_p.reshape(NQB, QB, D_MODEL), jnp.array(INV_PERM), axis=0
    ).reshape(1, SQ, D_MODEL)
    return out
